# baseline (speedup 1.0000x reference)
import jax
import jax.numpy as jnp
from jax import lax
from jax.experimental import pallas as pl
from jax.experimental.pallas import tpu as pltpu
from jax.experimental.pallas import tpu_sc as plsc

NUM_CLASSES = 1000
BATCH = 16384
NC, NS, L = 2, 16, 16
NW = NC * NS
COLS_W = BATCH // NW
R = 64
NCH = -(-NUM_CLASSES // R)
R_LAST = NUM_CLASSES - (NCH - 1) * R
NVREG = COLS_W // L


def _sc_body(idx_hbm, out_hbm, idx_v, buf0, buf1, sem0, sem1, sem_idx):
    wid = lax.axis_index("s") * NC + lax.axis_index("c")
    c0 = wid * COLS_W
    idx_copy = pltpu.make_async_copy(
        idx_hbm.at[pl.ds(c0, COLS_W)], idx_v, sem_idx
    )
    idx_copy.start()

    bufs = (buf0, buf1)
    sems = (sem0, sem1)
    zeros16 = jnp.zeros((L,), jnp.float32)
    ones16 = jnp.ones((L,), jnp.float32)
    lane_iota = lax.iota(jnp.int32, L)

    def zero_buf(buf):
        def zbody(r, _):
            def zcol(g, _):
                for u in range(4):
                    buf[r, pl.ds(g * (4 * L) + u * L, L)] = zeros16
                return 0

            lax.fori_loop(0, COLS_W // (4 * L), zcol, 0)
            return 0

        lax.fori_loop(0, R, zbody, 0)

    def in_range(row, n):
        return plsc.bitcast(row, jnp.uint32) < jnp.uint32(n)

    def process_chunk(i, b, rows_new, first):
        buf = bufs[b]
        rbase_new = i * R

        if not first:
            pltpu.make_async_copy(
                buf, out_hbm.at[pl.ds(0, R), pl.ds(c0, COLS_W)], sems[b]
            ).wait()

        def scan_body(g, _):
            off = g * L
            idxv = idx_v[pl.ds(off, L)]
            col = lane_iota + off
            new_row = idxv - rbase_new
            old_row = new_row + 2 * R
            plsc.store_scatter(
                buf, [old_row, col], zeros16, mask=in_range(old_row, R)
            )
            plsc.store_scatter(
                buf, [new_row, col], ones16, mask=in_range(new_row, rows_new)
            )
            return 0

        lax.fori_loop(0, NVREG, scan_body, 0)

        src = buf if rows_new == R else buf.at[pl.ds(0, rows_new)]
        dst = out_hbm.at[pl.ds(rbase_new, rows_new), pl.ds(c0, COLS_W)]
        pltpu.async_copy(src, dst, sems[b])

    zero_buf(buf0)
    idx_copy.wait()
    process_chunk(0, 0, R, True)
    zero_buf(buf1)
    process_chunk(1, 1, R, True)

    def body(j, _):
        for b in range(2):
            process_chunk(j * 2 + b, b, R, False)
        return 0

    lax.fori_loop(1, (NCH - 2) // 2, body, 0)
    process_chunk(NCH - 2, 0, R, False)
    process_chunk(NCH - 1, 1, R_LAST, False)

    pltpu.make_async_copy(
        bufs[0], out_hbm.at[pl.ds(0, R), pl.ds(c0, COLS_W)], sems[0]
    ).wait()
    pltpu.make_async_copy(
        bufs[1].at[pl.ds(0, R_LAST)],
        out_hbm.at[pl.ds(0, R_LAST), pl.ds(c0, COLS_W)],
        sems[1],
    ).wait()


def kernel(indices):
    k = pl.kernel(
        _sc_body,
        out_type=jax.ShapeDtypeStruct((NUM_CLASSES, BATCH), jnp.float32),
        mesh=plsc.VectorSubcoreMesh(
            core_axis_name="c", subcore_axis_name="s",
            num_cores=NC, num_subcores=NS,
        ),
        scratch_types=[
            pltpu.VMEM((COLS_W,), jnp.int32),
            pltpu.VMEM((R, COLS_W), jnp.float32),
            pltpu.VMEM((R, COLS_W), jnp.float32),
            pltpu.SemaphoreType.DMA,
            pltpu.SemaphoreType.DMA,
            pltpu.SemaphoreType.DMA,
        ],
        compiler_params=pltpu.CompilerParams(
            needs_layout_passes=False,
            use_tc_tiling_on_sc=True,
            disable_bounds_checks=True,
            disable_semaphore_checks=True,
            skip_device_barrier=True,
        ),
    )
    return k(indices.astype(jnp.int32)).T
